# baseline (speedup 1.0000x reference)
import functools

import jax
import jax.numpy as jnp
from jax.experimental import pallas as pl
from jax.experimental.pallas import tpu as pltpu


def _update_kernel(idx_ref, m_ref, mk_ref, mv_ref, out_ref, *, bs, K):
    b = pl.program_id(0)
    j = pl.program_id(1)
    out_ref[...] = m_ref[...]
    base = j * bs
    for k in range(K):
        idx = idx_ref[b, k]
        in_block = jnp.logical_and(idx >= base, idx < base + bs)

        @pl.when(in_block)
        def _():
            local = idx - base
            mk = mk_ref[0, idx, :]
            mv = mv_ref[0, idx, :]
            upd = mk[:, None] * mv[None, :]
            cur = out_ref[0, pl.ds(local, 1), :, :]
            out_ref[0, pl.ds(local, 1), :, :] = cur + upd[None, :, :]


@jax.jit
def kernel(M, M_k, M_v, indices_update):
    B, N, H, _ = M.shape
    K = indices_update.shape[1]
    bs = 64
    idx = indices_update.astype(jnp.int32)

    grid = (B, N // bs)
    out = pl.pallas_call(
        functools.partial(_update_kernel, bs=bs, K=K),
        grid_spec=pltpu.PrefetchScalarGridSpec(
            num_scalar_prefetch=1,
            grid=grid,
            in_specs=[
                pl.BlockSpec((1, bs, H, H), lambda b, j, idx_ref: (b, j, 0, 0)),
                pl.BlockSpec((1, N, H), lambda b, j, idx_ref: (b, 0, 0)),
                pl.BlockSpec((1, N, H), lambda b, j, idx_ref: (b, 0, 0)),
            ],
            out_specs=pl.BlockSpec((1, bs, H, H), lambda b, j, idx_ref: (b, j, 0, 0)),
        ),
        out_shape=jax.ShapeDtypeStruct(M.shape, M.dtype),
    )(idx, M, M_k, M_v)
    return out
